# baseline (speedup 1.0000x reference)
import functools

import jax
import jax.numpy as jnp
from jax import lax
from jax.experimental import pallas as pl
from jax.experimental.pallas import tpu as pltpu
from jax.experimental.pallas import tpu_sc as plsc

NV = 6890
NN = 689
K = 3
NB = 18

NWORK = 32
VPT = 224
VP = NWORK * VPT
WGRP = VPT // 16
AGRP = 2
NP = 704
NRP = 1024
NP3 = 3 * NP
VP3 = 3 * VP
RINGP = NWORK * AGRP * 16 * NB

_mesh = plsc.VectorSubcoreMesh(core_axis_name="c", subcore_axis_name="s")
_sc_params = pltpu.CompilerParams(needs_layout_passes=False)


def _tc_body(r_ref, out_ref):
    x = r_ref[0]
    y = r_ref[1]
    z = r_ref[2]
    xa = x + 1e-8
    ya = y + 1e-8
    za = z + 1e-8
    ang = jnp.sqrt(xa * xa + ya * ya + za * za)
    ax = x / ang
    ay = y / ang
    az = z / ang
    c = jnp.cos(ang)
    s = jnp.sin(ang)
    cc = 1.0 - c
    out_ref[0] = c + cc * ax * ax
    out_ref[1] = cc * ax * ay - s * az
    out_ref[2] = cc * ax * az + s * ay
    out_ref[3] = cc * ax * ay + s * az
    out_ref[4] = c + cc * ay * ay
    out_ref[5] = cc * ay * az - s * ax
    out_ref[6] = cc * ax * az - s * ay
    out_ref[7] = cc * ay * az + s * ax
    out_ref[8] = c + cc * az * az


_tc_rot = pl.pallas_call(
    _tc_body,
    out_shape=jax.ShapeDtypeStruct((9, 8, 128), jnp.float32),
)


@functools.partial(
    pl.kernel,
    mesh=_mesh,
    out_type=(
        jax.ShapeDtypeStruct((VP3,), jnp.float32),
        jax.ShapeDtypeStruct((NWORK * 16,), jnp.float32),
    ),
    compiler_params=_sc_params,
    scratch_types=[
        pltpu.VMEM((VP3,), jnp.float32),
        pltpu.VMEM((VPT * 3,), jnp.float32),
        pltpu.VMEM((VPT * 3,), jnp.int32),
        pltpu.VMEM((9 * NRP,), jnp.float32),
        pltpu.VMEM((NP3,), jnp.float32),
        pltpu.VMEM((NP,), jnp.int32),
        pltpu.VMEM((AGRP * 16 * NB,), jnp.int32),
        pltpu.VMEM((9 * NP,), jnp.float32),
        pltpu.VMEM((VPT * 3,), jnp.float32),
        pltpu.VMEM((16,), jnp.float32),
    ],
)
def _dgraph(v_hbm, w_hbm, ix_hbm, rt_hbm, tv_hbm, nidx_hbm, ring_hbm,
            warp_hbm, loss_hbm,
            v_v, w_v, ix_v, rt_v, tv_v, nidx_v, ring_v, aux_v, out_v,
            loss_v):
    wid = lax.axis_index("s") * 2 + lax.axis_index("c")
    base = wid * VPT

    pltpu.sync_copy(v_hbm, v_v)
    pltpu.sync_copy(w_hbm.at[pl.ds(base * 3, VPT * 3)], w_v)
    pltpu.sync_copy(ix_hbm.at[pl.ds(base * 3, VPT * 3)], ix_v)
    pltpu.sync_copy(rt_hbm, rt_v)
    pltpu.sync_copy(tv_hbm, tv_v)
    pltpu.sync_copy(nidx_hbm, nidx_v)
    pltpu.sync_copy(
        ring_hbm.at[pl.ds(wid * AGRP * 16 * NB, AGRP * 16 * NB)], ring_v)

    ids = lax.iota(jnp.int32, 16)

    def build_group(g, carry):
        nids = g * 16 + ids
        n3 = nids * 3
        sl = pl.ds(g * 16, 16)
        r = [rt_v[pl.ds(t * NRP + g * 16, 16)] for t in range(9)]
        j = plsc.load_gather(nidx_v, [nids])
        j3 = j * 3
        nx = plsc.load_gather(v_v, [j3])
        ny = plsc.load_gather(v_v, [j3 + 1])
        nz = plsc.load_gather(v_v, [j3 + 2])
        pmx = nx + plsc.load_gather(tv_v, [n3])
        pmy = ny + plsc.load_gather(tv_v, [n3 + 1])
        pmz = nz + plsc.load_gather(tv_v, [n3 + 2])
        aux_v[sl] = pmx - (r[0] * nx + r[1] * ny + r[2] * nz)
        aux_v[pl.ds(NP + g * 16, 16)] = pmy - (
            r[3] * nx + r[4] * ny + r[5] * nz)
        aux_v[pl.ds(2 * NP + g * 16, 16)] = pmz - (
            r[6] * nx + r[7] * ny + r[8] * nz)
        aux_v[pl.ds(3 * NP + g * 16, 16)] = pmx
        aux_v[pl.ds(4 * NP + g * 16, 16)] = pmy
        aux_v[pl.ds(5 * NP + g * 16, 16)] = pmz
        aux_v[pl.ds(6 * NP + g * 16, 16)] = nx
        aux_v[pl.ds(7 * NP + g * 16, 16)] = ny
        aux_v[pl.ds(8 * NP + g * 16, 16)] = nz
        return carry

    lax.fori_loop(0, NP // 16, build_group, 0)

    def warp_group(g, carry):
        l3 = (g * 16 + ids) * 3
        v3 = base * 3 + l3
        vx = plsc.load_gather(v_v, [v3])
        vy = plsc.load_gather(v_v, [v3 + 1])
        vz = plsc.load_gather(v_v, [v3 + 2])
        acc = [jnp.zeros((16,), jnp.float32) for _ in range(12)]
        for k in range(K):
            j = plsc.load_gather(ix_v, [l3 + k])
            w = plsc.load_gather(w_v, [l3 + k])
            for t in range(9):
                acc[t] = acc[t] + w * plsc.load_gather(rt_v, [j + t * NRP])
            for t in range(3):
                acc[9 + t] = acc[9 + t] + w * plsc.load_gather(
                    aux_v, [j + t * NP])
        plsc.store_scatter(out_v, [l3],
                           acc[0] * vx + acc[1] * vy + acc[2] * vz + acc[9])
        plsc.store_scatter(out_v, [l3 + 1],
                           acc[3] * vx + acc[4] * vy + acc[5] * vz + acc[10])
        plsc.store_scatter(out_v, [l3 + 2],
                           acc[6] * vx + acc[7] * vy + acc[8] * vz + acc[11])
        return carry

    lax.fori_loop(0, WGRP, warp_group, 0)
    pltpu.sync_copy(out_v, warp_hbm.at[pl.ds(base * 3, VPT * 3)])

    acc_loss = jnp.zeros((16,), jnp.float32)
    for gg in range(AGRP):
        gbase = (wid * AGRP + gg) * 16
        gclamp = jnp.minimum(gbase, NP - 16)
        r = [rt_v[pl.ds(t * NRP + gbase, 16)] for t in range(9)]
        pm = [aux_v[pl.ds((3 + ci) * NP + gclamp, 16)] for ci in range(3)]
        nn = [aux_v[pl.ds((6 + ci) * NP + gclamp, 16)] for ci in range(3)]
        valid = (gbase + ids) < NN
        for h in range(NB):
            m = plsc.load_gather(ring_v, [(gg * 16 + ids) * NB + h])
            nm = [plsc.load_gather(aux_v, [m + (6 + ci) * NP])
                  for ci in range(3)]
            pmm = [plsc.load_gather(aux_v, [m + (3 + ci) * NP])
                   for ci in range(3)]
            dx = nn[0] - nm[0]
            dy = nn[1] - nm[1]
            dz = nn[2] - nm[2]
            ex = pm[0] - pmm[0] - (r[0] * dx + r[1] * dy + r[2] * dz)
            ey = pm[1] - pmm[1] - (r[3] * dx + r[4] * dy + r[5] * dz)
            ez = pm[2] - pmm[2] - (r[6] * dx + r[7] * dy + r[8] * dz)
            e2 = ex * ex + ey * ey + ez * ez
            acc_loss = acc_loss + jnp.where(valid, e2, 0.0)
    loss_v[...] = acc_loss
    pltpu.sync_copy(loss_v, loss_hbm.at[pl.ds(wid * 16, 16)])


def _padto(x, n):
    return jnp.concatenate([x, jnp.zeros((n - x.shape[0],), x.dtype)])


def kernel(vertices, opt_d_rotations, opt_d_translations, weights, nodes_idx,
           influence_nodes_idx, one_ring_neigh):
    i32 = jnp.int32
    f32 = jnp.float32
    vflat = _padto(vertices.reshape(-1), VP3)
    wflat = _padto(weights.reshape(-1), VP3)
    ixflat = _padto(influence_nodes_idx.astype(i32).reshape(-1), VP3)
    rv = jnp.zeros((3, NRP), f32).at[:, :NN].set(opt_d_rotations[0].T)
    tvflat = _padto(opt_d_translations.reshape(-1), NP3)
    nidx = _padto(nodes_idx.astype(i32), NP)
    ringflat = _padto(one_ring_neigh.astype(i32).reshape(-1), RINGP)

    rtab = _tc_rot(rv.reshape(3, 8, 128))
    warp, loss_part = _dgraph(vflat, wflat, ixflat, rtab.reshape(-1),
                              tvflat, nidx, ringflat)
    warped = warp[:NV * 3].reshape(1, NV, 3)
    arap = jnp.sum(loss_part) / f32(NN)
    return warped, arap


# speedup vs baseline: 1.1054x; 1.1054x over previous
import functools

import jax
import jax.numpy as jnp
from jax import lax
from jax.experimental import pallas as pl
from jax.experimental.pallas import tpu as pltpu
from jax.experimental.pallas import tpu_sc as plsc

NV = 6890
NN = 689
K = 3
NB = 18

NWORK = 32
VPT = 224
VP = NWORK * VPT
WGRP = VPT // 16
AGRP = 2
NRP = 1024
VP3 = 3 * VP
RINGP = NWORK * AGRP * 16 * NB

_mesh = plsc.VectorSubcoreMesh(core_axis_name="c", subcore_axis_name="s")
_sc_params = pltpu.CompilerParams(needs_layout_passes=False)


def _wid():
    return lax.axis_index("s") * 2 + lax.axis_index("c")


@functools.partial(
    pl.kernel,
    mesh=_mesh,
    out_type=jax.ShapeDtypeStruct((3 * NRP,), jnp.float32),
    compiler_params=_sc_params,
    scratch_types=[
        pltpu.VMEM((VP3,), jnp.float32),
        pltpu.VMEM((32,), jnp.int32),
        pltpu.VMEM((3 * 32,), jnp.float32),
    ],
)
def _gather_nodes(v_hbm, idx_hbm, out_hbm, v_v, idx_v, out_v):
    wid = _wid()
    pltpu.sync_copy(v_hbm, v_v)
    pltpu.sync_copy(idx_hbm.at[pl.ds(wid * 32, 32)], idx_v)
    for gg in range(2):
        j3 = idx_v[pl.ds(gg * 16, 16)] * 3
        for ci in range(3):
            out_v[pl.ds(ci * 32 + gg * 16, 16)] = plsc.load_gather(
                v_v, [j3 + ci])
    for ci in range(3):
        pltpu.sync_copy(out_v.at[pl.ds(ci * 32, 32)],
                        out_hbm.at[pl.ds(ci * NRP + wid * 32, 32)])


def _tc_body(r_ref, t_ref, n_ref, out_ref):
    x = r_ref[0]
    y = r_ref[1]
    z = r_ref[2]
    xa = x + 1e-8
    ya = y + 1e-8
    za = z + 1e-8
    ang = jnp.sqrt(xa * xa + ya * ya + za * za)
    ax = x / ang
    ay = y / ang
    az = z / ang
    c = jnp.cos(ang)
    s = jnp.sin(ang)
    cc = 1.0 - c
    r00 = c + cc * ax * ax
    r01 = cc * ax * ay - s * az
    r02 = cc * ax * az + s * ay
    r10 = cc * ax * ay + s * az
    r11 = c + cc * ay * ay
    r12 = cc * ay * az - s * ax
    r20 = cc * ax * az - s * ay
    r21 = cc * ay * az + s * ax
    r22 = c + cc * az * az
    nx = n_ref[0]
    ny = n_ref[1]
    nz = n_ref[2]
    pmx = nx + t_ref[0]
    pmy = ny + t_ref[1]
    pmz = nz + t_ref[2]
    out_ref[0] = r00
    out_ref[1] = r01
    out_ref[2] = r02
    out_ref[3] = r10
    out_ref[4] = r11
    out_ref[5] = r12
    out_ref[6] = r20
    out_ref[7] = r21
    out_ref[8] = r22
    out_ref[9] = pmx - (r00 * nx + r01 * ny + r02 * nz)
    out_ref[10] = pmy - (r10 * nx + r11 * ny + r12 * nz)
    out_ref[11] = pmz - (r20 * nx + r21 * ny + r22 * nz)
    out_ref[12] = pmx
    out_ref[13] = pmy
    out_ref[14] = pmz
    out_ref[15] = nx
    out_ref[16] = ny
    out_ref[17] = nz


_tc_table = pl.pallas_call(
    _tc_body,
    out_shape=jax.ShapeDtypeStruct((18, 8, 128), jnp.float32),
)


@functools.partial(
    pl.kernel,
    mesh=_mesh,
    out_type=jax.ShapeDtypeStruct((VP3,), jnp.float32),
    compiler_params=_sc_params,
    scratch_types=[
        pltpu.VMEM((VPT * 3,), jnp.float32),
        pltpu.VMEM((VPT * 3,), jnp.float32),
        pltpu.VMEM((VPT * 3,), jnp.int32),
        pltpu.VMEM((12 * NRP,), jnp.float32),
        pltpu.VMEM((VPT * 3,), jnp.float32),
        pltpu.SemaphoreType.DMA,
    ],
)
def _warp(v_hbm, w_hbm, ix_hbm, tab_hbm, warp_hbm,
          v_v, w_v, ix_v, tab_v, out_v, sem):
    wid = _wid()
    b3 = wid * VPT * 3
    cps = [
        pltpu.async_copy(v_hbm.at[pl.ds(b3, VPT * 3)], v_v, sem),
        pltpu.async_copy(w_hbm.at[pl.ds(b3, VPT * 3)], w_v, sem),
        pltpu.async_copy(ix_hbm.at[pl.ds(b3, VPT * 3)], ix_v, sem),
        pltpu.async_copy(tab_hbm.at[pl.ds(0, 12 * NRP)], tab_v, sem),
    ]
    for cp in cps:
        cp.wait()
    ids = lax.iota(jnp.int32, 16)

    def warp_group(g, carry):
        l3 = (g * 16 + ids) * 3
        vx = plsc.load_gather(v_v, [l3])
        vy = plsc.load_gather(v_v, [l3 + 1])
        vz = plsc.load_gather(v_v, [l3 + 2])
        acc = [jnp.zeros((16,), jnp.float32) for _ in range(12)]
        for k in range(K):
            j = plsc.load_gather(ix_v, [l3 + k])
            w = plsc.load_gather(w_v, [l3 + k])
            for t in range(12):
                acc[t] = acc[t] + w * plsc.load_gather(tab_v, [j + t * NRP])
        plsc.store_scatter(out_v, [l3],
                           acc[0] * vx + acc[1] * vy + acc[2] * vz + acc[9])
        plsc.store_scatter(out_v, [l3 + 1],
                           acc[3] * vx + acc[4] * vy + acc[5] * vz + acc[10])
        plsc.store_scatter(out_v, [l3 + 2],
                           acc[6] * vx + acc[7] * vy + acc[8] * vz + acc[11])
        return carry

    lax.fori_loop(0, WGRP, warp_group, 0)
    pltpu.async_copy(out_v, warp_hbm.at[pl.ds(b3, VPT * 3)], sem).wait()


@functools.partial(
    pl.kernel,
    mesh=_mesh,
    out_type=jax.ShapeDtypeStruct((NWORK * 16,), jnp.float32),
    compiler_params=_sc_params,
    scratch_types=[
        pltpu.VMEM((18 * NRP,), jnp.float32),
        pltpu.VMEM((AGRP * 16 * NB,), jnp.int32),
        pltpu.VMEM((16,), jnp.float32),
        pltpu.SemaphoreType.DMA,
    ],
)
def _arap(tab_hbm, ring_hbm, loss_hbm, tab_v, ring_v, loss_v, sem):
    wid = _wid()
    cps = [
        pltpu.async_copy(tab_hbm, tab_v, sem),
        pltpu.async_copy(
            ring_hbm.at[pl.ds(wid * AGRP * 16 * NB, AGRP * 16 * NB)],
            ring_v, sem),
    ]
    for cp in cps:
        cp.wait()
    ids = lax.iota(jnp.int32, 16)
    acc_loss = jnp.zeros((16,), jnp.float32)
    for gg in range(AGRP):
        gbase = (wid * AGRP + gg) * 16
        r = [tab_v[pl.ds(t * NRP + gbase, 16)] for t in range(9)]
        pm = [tab_v[pl.ds((12 + ci) * NRP + gbase, 16)] for ci in range(3)]
        nn = [tab_v[pl.ds((15 + ci) * NRP + gbase, 16)] for ci in range(3)]
        valid = (gbase + ids) < NN
        for h in range(NB):
            m = plsc.load_gather(ring_v, [(gg * 16 + ids) * NB + h])
            nm = [plsc.load_gather(tab_v, [m + (15 + ci) * NRP])
                  for ci in range(3)]
            pmm = [plsc.load_gather(tab_v, [m + (12 + ci) * NRP])
                   for ci in range(3)]
            dx = nn[0] - nm[0]
            dy = nn[1] - nm[1]
            dz = nn[2] - nm[2]
            ex = pm[0] - pmm[0] - (r[0] * dx + r[1] * dy + r[2] * dz)
            ey = pm[1] - pmm[1] - (r[3] * dx + r[4] * dy + r[5] * dz)
            ez = pm[2] - pmm[2] - (r[6] * dx + r[7] * dy + r[8] * dz)
            e2 = ex * ex + ey * ey + ez * ez
            acc_loss = acc_loss + jnp.where(valid, e2, 0.0)
    loss_v[...] = acc_loss
    pltpu.async_copy(loss_v, loss_hbm.at[pl.ds(wid * 16, 16)], sem).wait()


def _padto(x, n):
    return jnp.concatenate([x, jnp.zeros((n - x.shape[0],), x.dtype)])


def kernel(vertices, opt_d_rotations, opt_d_translations, weights, nodes_idx,
           influence_nodes_idx, one_ring_neigh):
    i32 = jnp.int32
    f32 = jnp.float32
    vflat = _padto(vertices.reshape(-1), VP3)
    wflat = _padto(weights.reshape(-1), VP3)
    ixflat = _padto(influence_nodes_idx.astype(i32).reshape(-1), VP3)
    nidx = _padto(nodes_idx.astype(i32), NRP)
    rv = jnp.zeros((3, NRP), f32).at[:, :NN].set(opt_d_rotations[0].T)
    tv = jnp.zeros((3, NRP), f32).at[:, :NN].set(opt_d_translations[0].T)
    ringflat = _padto(one_ring_neigh.astype(i32).reshape(-1), RINGP)

    nplanes = _gather_nodes(vflat, nidx)
    table = _tc_table(rv.reshape(3, 8, 128), tv.reshape(3, 8, 128),
                      nplanes.reshape(3, 8, 128))
    tab_flat = table.reshape(-1)
    warp = _warp(vflat, wflat, ixflat, tab_flat)
    loss_part = _arap(tab_flat, ringflat)
    warped = warp[:NV * 3].reshape(1, NV, 3)
    arap = jnp.sum(loss_part) / f32(NN)
    return warped, arap


# speedup vs baseline: 1.3391x; 1.2114x over previous
import functools

import jax
import jax.numpy as jnp
from jax import lax
from jax.experimental import pallas as pl
from jax.experimental.pallas import tpu as pltpu
from jax.experimental.pallas import tpu_sc as plsc

NV = 6890
NN = 689
K = 3
NB = 18

NNP = 1024
VP = 7168
NWORK = 32
VPT = VP // NWORK
WGRP = VPT // 16
AGRP = NNP // 16 // NWORK

_mesh = plsc.VectorSubcoreMesh(core_axis_name="c", subcore_axis_name="s")
_sc_params = pltpu.CompilerParams(needs_layout_passes=False)


def _wid():
    return lax.axis_index("s") * 2 + lax.axis_index("c")


@functools.partial(
    pl.kernel,
    mesh=_mesh,
    out_type=jax.ShapeDtypeStruct((3 * NNP,), jnp.float32),
    compiler_params=_sc_params,
    scratch_types=[
        pltpu.VMEM((3 * VP,), jnp.float32),
        pltpu.VMEM((32,), jnp.int32),
        pltpu.VMEM((3 * 32,), jnp.float32),
    ],
)
def _gather_nodes(v_hbm, idx_hbm, out_hbm, v_v, idx_v, out_v):
    wid = _wid()
    pltpu.sync_copy(v_hbm, v_v)
    pltpu.sync_copy(idx_hbm.at[pl.ds(wid * 32, 32)], idx_v)
    for gg in range(2):
        j = idx_v[pl.ds(gg * 16, 16)]
        for ci in range(3):
            out_v[pl.ds(ci * 32 + gg * 16, 16)] = plsc.load_gather(
                v_v, [j + ci * VP])
    for ci in range(3):
        pltpu.sync_copy(out_v.at[pl.ds(ci * 32, 32)],
                        out_hbm.at[pl.ds(ci * NNP + wid * 32, 32)])


def _tc_body(r_ref, t_ref, n_ref, out_ref):
    x = r_ref[0]
    y = r_ref[1]
    z = r_ref[2]
    xa = x + 1e-8
    ya = y + 1e-8
    za = z + 1e-8
    ang = jnp.sqrt(xa * xa + ya * ya + za * za)
    ax = x / ang
    ay = y / ang
    az = z / ang
    c = jnp.cos(ang)
    s = jnp.sin(ang)
    cc = 1.0 - c
    r00 = c + cc * ax * ax
    r01 = cc * ax * ay - s * az
    r02 = cc * ax * az + s * ay
    r10 = cc * ax * ay + s * az
    r11 = c + cc * ay * ay
    r12 = cc * ay * az - s * ax
    r20 = cc * ax * az - s * ay
    r21 = cc * ay * az + s * ax
    r22 = c + cc * az * az
    nx = n_ref[0]
    ny = n_ref[1]
    nz = n_ref[2]
    pmx = nx + t_ref[0]
    pmy = ny + t_ref[1]
    pmz = nz + t_ref[2]
    out_ref[0] = r00
    out_ref[1] = r01
    out_ref[2] = r02
    out_ref[3] = r10
    out_ref[4] = r11
    out_ref[5] = r12
    out_ref[6] = r20
    out_ref[7] = r21
    out_ref[8] = r22
    out_ref[9] = pmx - (r00 * nx + r01 * ny + r02 * nz)
    out_ref[10] = pmy - (r10 * nx + r11 * ny + r12 * nz)
    out_ref[11] = pmz - (r20 * nx + r21 * ny + r22 * nz)
    out_ref[12] = pmx
    out_ref[13] = pmy
    out_ref[14] = pmz
    out_ref[15] = nx
    out_ref[16] = ny
    out_ref[17] = nz


_tc_table = pl.pallas_call(
    _tc_body,
    out_shape=jax.ShapeDtypeStruct((18, 8, 128), jnp.float32),
)


@functools.partial(
    pl.kernel,
    mesh=_mesh,
    out_type=(
        jax.ShapeDtypeStruct((3 * VP,), jnp.float32),
        jax.ShapeDtypeStruct((NWORK * 16,), jnp.float32),
    ),
    compiler_params=_sc_params,
    scratch_types=[
        pltpu.VMEM((18 * NNP,), jnp.float32),
        pltpu.VMEM((3 * VPT,), jnp.float32),
        pltpu.VMEM((3 * VPT,), jnp.float32),
        pltpu.VMEM((3 * VPT,), jnp.int32),
        pltpu.VMEM((AGRP * NB * 16,), jnp.int32),
        pltpu.VMEM((3 * VPT,), jnp.float32),
        pltpu.VMEM((16,), jnp.float32),
    ],
)
def _warp_arap(v_hbm, w_hbm, ix_hbm, tab_hbm, ring_hbm, warp_hbm, loss_hbm,
               tab_v, v_v, w_v, ix_v, ring_v, wout_v, loss_v):
    wid = _wid()
    base = wid * VPT
    pltpu.sync_copy(tab_hbm, tab_v)
    for ci in range(3):
        sl_h = pl.ds(ci * VP + base, VPT)
        sl_v = pl.ds(ci * VPT, VPT)
        pltpu.sync_copy(v_hbm.at[sl_h], v_v.at[sl_v])
        pltpu.sync_copy(w_hbm.at[sl_h], w_v.at[sl_v])
        pltpu.sync_copy(ix_hbm.at[sl_h], ix_v.at[sl_v])
    pltpu.sync_copy(ring_hbm.at[pl.ds(wid * AGRP * NB * 16, AGRP * NB * 16)],
                    ring_v)

    def warp_group(g, carry):
        vx = v_v[pl.ds(g * 16, 16)]
        vy = v_v[pl.ds(VPT + g * 16, 16)]
        vz = v_v[pl.ds(2 * VPT + g * 16, 16)]
        acc = [jnp.zeros((16,), jnp.float32) for _ in range(12)]
        for k in range(K):
            j = ix_v[pl.ds(k * VPT + g * 16, 16)]
            w = w_v[pl.ds(k * VPT + g * 16, 16)]
            for t in range(12):
                acc[t] = acc[t] + w * plsc.load_gather(tab_v, [j + t * NNP])
        wout_v[pl.ds(g * 16, 16)] = (
            acc[0] * vx + acc[1] * vy + acc[2] * vz + acc[9])
        wout_v[pl.ds(VPT + g * 16, 16)] = (
            acc[3] * vx + acc[4] * vy + acc[5] * vz + acc[10])
        wout_v[pl.ds(2 * VPT + g * 16, 16)] = (
            acc[6] * vx + acc[7] * vy + acc[8] * vz + acc[11])
        return carry

    lax.fori_loop(0, WGRP, warp_group, 0)
    for ci in range(3):
        pltpu.sync_copy(wout_v.at[pl.ds(ci * VPT, VPT)],
                        warp_hbm.at[pl.ds(ci * VP + base, VPT)])

    ids = lax.iota(jnp.int32, 16)
    acc_loss = jnp.zeros((16,), jnp.float32)
    for gg in range(AGRP):
        gbase = (wid * AGRP + gg) * 16
        r = [tab_v[pl.ds(t * NNP + gbase, 16)] for t in range(9)]
        pm = [tab_v[pl.ds((12 + ci) * NNP + gbase, 16)] for ci in range(3)]
        nn = [tab_v[pl.ds((15 + ci) * NNP + gbase, 16)] for ci in range(3)]
        valid = (gbase + ids) < NN
        for h in range(NB):
            m = ring_v[pl.ds((gg * NB + h) * 16, 16)]
            nm = [plsc.load_gather(tab_v, [m + (15 + ci) * NNP])
                  for ci in range(3)]
            pmm = [plsc.load_gather(tab_v, [m + (12 + ci) * NNP])
                   for ci in range(3)]
            dx = nn[0] - nm[0]
            dy = nn[1] - nm[1]
            dz = nn[2] - nm[2]
            ex = pm[0] - pmm[0] - (r[0] * dx + r[1] * dy + r[2] * dz)
            ey = pm[1] - pmm[1] - (r[3] * dx + r[4] * dy + r[5] * dz)
            ez = pm[2] - pmm[2] - (r[6] * dx + r[7] * dy + r[8] * dz)
            e2 = ex * ex + ey * ey + ez * ez
            acc_loss = acc_loss + jnp.where(valid, e2, 0.0)
    loss_v[...] = acc_loss
    pltpu.sync_copy(loss_v, loss_hbm.at[pl.ds(wid * 16, 16)])


def kernel(vertices, opt_d_rotations, opt_d_translations, weights, nodes_idx,
           influence_nodes_idx, one_ring_neigh):
    f32 = jnp.float32
    i32 = jnp.int32
    vp = jnp.zeros((3, VP), f32).at[:, :NV].set(vertices.T).reshape(-1)
    wp = jnp.zeros((3, VP), f32).at[:, :NV].set(weights.T).reshape(-1)
    ip = jnp.zeros((3, VP), i32).at[:, :NV].set(
        influence_nodes_idx.T.astype(i32)).reshape(-1)
    nidx = jnp.zeros((NNP,), i32).at[:NN].set(nodes_idx.astype(i32))
    rv = jnp.zeros((3, NNP), f32).at[:, :NN].set(opt_d_rotations[0].T)
    tv = jnp.zeros((3, NNP), f32).at[:, :NN].set(opt_d_translations[0].T)
    ring = jnp.zeros((NWORK * AGRP * 16, NB), i32).at[:NN].set(
        one_ring_neigh.astype(i32))
    ring = ring.reshape(NWORK * AGRP, 16, NB).transpose(0, 2, 1).reshape(-1)

    nplanes = _gather_nodes(vp, nidx)
    table = _tc_table(rv.reshape(3, 8, 128), tv.reshape(3, 8, 128),
                      nplanes.reshape(3, 8, 128))
    warp, loss_part = _warp_arap(vp, wp, ip, table.reshape(-1), ring)
    warped = warp.reshape(3, VP)[:, :NV].T[None]
    arap = jnp.sum(loss_part) / f32(NN)
    return warped, arap


# speedup vs baseline: 1.4883x; 1.1114x over previous
import functools

import jax
import jax.numpy as jnp
from jax import lax
from jax.experimental import pallas as pl
from jax.experimental.pallas import tpu as pltpu
from jax.experimental.pallas import tpu_sc as plsc

NV = 6890
NN = 689
K = 3
NB = 18

NNP = 1024
VP = 7168
NWORK = 32
VPT = VP // NWORK
WGRP = VPT // 16
AGRP = NNP // 16 // NWORK

_mesh = plsc.VectorSubcoreMesh(core_axis_name="c", subcore_axis_name="s")
_sc_params = pltpu.CompilerParams(needs_layout_passes=False)


def _wid():
    return lax.axis_index("s") * 2 + lax.axis_index("c")


@functools.partial(
    pl.kernel,
    mesh=_mesh,
    out_type=jax.ShapeDtypeStruct((3 * NNP,), jnp.float32),
    compiler_params=_sc_params,
    scratch_types=[
        pltpu.VMEM((3 * VP,), jnp.float32),
        pltpu.VMEM((32,), jnp.int32),
        pltpu.VMEM((3 * 32,), jnp.float32),
    ],
)
def _gather_nodes(v_hbm, idx_hbm, out_hbm, v_v, idx_v, out_v):
    wid = _wid()
    pltpu.sync_copy(v_hbm, v_v)
    pltpu.sync_copy(idx_hbm.at[pl.ds(wid * 32, 32)], idx_v)
    for gg in range(2):
        j = idx_v[pl.ds(gg * 16, 16)]
        for ci in range(3):
            out_v[pl.ds(ci * 32 + gg * 16, 16)] = plsc.load_gather(
                v_v, [j + ci * VP])
    for ci in range(3):
        pltpu.sync_copy(out_v.at[pl.ds(ci * 32, 32)],
                        out_hbm.at[pl.ds(ci * NNP + wid * 32, 32)])


def _tc_body(r_ref, t_ref, n_ref, out_ref):
    x = r_ref[0]
    y = r_ref[1]
    z = r_ref[2]
    xa = x + 1e-8
    ya = y + 1e-8
    za = z + 1e-8
    ang = jnp.sqrt(xa * xa + ya * ya + za * za)
    ax = x / ang
    ay = y / ang
    az = z / ang
    c = jnp.cos(ang)
    s = jnp.sin(ang)
    cc = 1.0 - c
    r00 = c + cc * ax * ax
    r01 = cc * ax * ay - s * az
    r02 = cc * ax * az + s * ay
    r10 = cc * ax * ay + s * az
    r11 = c + cc * ay * ay
    r12 = cc * ay * az - s * ax
    r20 = cc * ax * az - s * ay
    r21 = cc * ay * az + s * ax
    r22 = c + cc * az * az
    nx = n_ref[0]
    ny = n_ref[1]
    nz = n_ref[2]
    pmx = nx + t_ref[0]
    pmy = ny + t_ref[1]
    pmz = nz + t_ref[2]
    out_ref[0] = r00
    out_ref[1] = r01
    out_ref[2] = r02
    out_ref[3] = r10
    out_ref[4] = r11
    out_ref[5] = r12
    out_ref[6] = r20
    out_ref[7] = r21
    out_ref[8] = r22
    out_ref[9] = pmx - (r00 * nx + r01 * ny + r02 * nz)
    out_ref[10] = pmy - (r10 * nx + r11 * ny + r12 * nz)
    out_ref[11] = pmz - (r20 * nx + r21 * ny + r22 * nz)
    out_ref[12] = pmx
    out_ref[13] = pmy
    out_ref[14] = pmz
    out_ref[15] = nx
    out_ref[16] = ny
    out_ref[17] = nz


_tc_table = pl.pallas_call(
    _tc_body,
    out_shape=jax.ShapeDtypeStruct((18, 8, 128), jnp.float32),
)


@functools.partial(
    pl.kernel,
    mesh=_mesh,
    out_type=(
        jax.ShapeDtypeStruct((3 * VP,), jnp.float32),
        jax.ShapeDtypeStruct((NWORK * 16,), jnp.float32),
    ),
    compiler_params=_sc_params,
    scratch_types=[
        pltpu.VMEM((18 * NNP,), jnp.float32),
        pltpu.VMEM((3 * VPT,), jnp.float32),
        pltpu.VMEM((3 * VPT,), jnp.float32),
        pltpu.VMEM((3 * VPT,), jnp.int32),
        pltpu.VMEM((AGRP * NB * 16,), jnp.int32),
        pltpu.VMEM((3 * VPT,), jnp.float32),
        pltpu.VMEM((16,), jnp.float32),
        pltpu.SemaphoreType.DMA,
    ],
)
def _warp_arap(v_hbm, w_hbm, ix_hbm, tab_hbm, ring_hbm, warp_hbm, loss_hbm,
               tab_v, v_v, w_v, ix_v, ring_v, wout_v, loss_v, sem):
    wid = _wid()
    base = wid * VPT
    cps = [pltpu.async_copy(tab_hbm, tab_v, sem)]
    for ci in range(3):
        sl_h = pl.ds(ci * VP + base, VPT)
        sl_v = pl.ds(ci * VPT, VPT)
        cps.append(pltpu.async_copy(v_hbm.at[sl_h], v_v.at[sl_v], sem))
        cps.append(pltpu.async_copy(w_hbm.at[sl_h], w_v.at[sl_v], sem))
        cps.append(pltpu.async_copy(ix_hbm.at[sl_h], ix_v.at[sl_v], sem))
    cps.append(pltpu.async_copy(
        ring_hbm.at[pl.ds(wid * AGRP * NB * 16, AGRP * NB * 16)], ring_v,
        sem))
    for cp in cps:
        cp.wait()

    def warp_group(g, carry):
        vx = v_v[pl.ds(g * 16, 16)]
        vy = v_v[pl.ds(VPT + g * 16, 16)]
        vz = v_v[pl.ds(2 * VPT + g * 16, 16)]
        acc = [jnp.zeros((16,), jnp.float32) for _ in range(12)]
        for k in range(K):
            j = ix_v[pl.ds(k * VPT + g * 16, 16)]
            w = w_v[pl.ds(k * VPT + g * 16, 16)]
            for t in range(12):
                acc[t] = acc[t] + w * plsc.load_gather(tab_v, [j + t * NNP])
        wout_v[pl.ds(g * 16, 16)] = (
            acc[0] * vx + acc[1] * vy + acc[2] * vz + acc[9])
        wout_v[pl.ds(VPT + g * 16, 16)] = (
            acc[3] * vx + acc[4] * vy + acc[5] * vz + acc[10])
        wout_v[pl.ds(2 * VPT + g * 16, 16)] = (
            acc[6] * vx + acc[7] * vy + acc[8] * vz + acc[11])
        return carry

    lax.fori_loop(0, WGRP, warp_group, 0)
    for ci in range(3):
        pltpu.sync_copy(wout_v.at[pl.ds(ci * VPT, VPT)],
                        warp_hbm.at[pl.ds(ci * VP + base, VPT)])

    ids = lax.iota(jnp.int32, 16)
    acc_loss = jnp.zeros((16,), jnp.float32)
    for gg in range(AGRP):
        gbase = (wid * AGRP + gg) * 16
        r = [tab_v[pl.ds(t * NNP + gbase, 16)] for t in range(9)]
        pm = [tab_v[pl.ds((12 + ci) * NNP + gbase, 16)] for ci in range(3)]
        nn = [tab_v[pl.ds((15 + ci) * NNP + gbase, 16)] for ci in range(3)]
        valid = (gbase + ids) < NN
        for h in range(NB):
            m = ring_v[pl.ds((gg * NB + h) * 16, 16)]
            nm = [plsc.load_gather(tab_v, [m + (15 + ci) * NNP])
                  for ci in range(3)]
            pmm = [plsc.load_gather(tab_v, [m + (12 + ci) * NNP])
                   for ci in range(3)]
            dx = nn[0] - nm[0]
            dy = nn[1] - nm[1]
            dz = nn[2] - nm[2]
            ex = pm[0] - pmm[0] - (r[0] * dx + r[1] * dy + r[2] * dz)
            ey = pm[1] - pmm[1] - (r[3] * dx + r[4] * dy + r[5] * dz)
            ez = pm[2] - pmm[2] - (r[6] * dx + r[7] * dy + r[8] * dz)
            e2 = ex * ex + ey * ey + ez * ez
            acc_loss = acc_loss + jnp.where(valid, e2, 0.0)
    loss_v[...] = acc_loss
    pltpu.sync_copy(loss_v, loss_hbm.at[pl.ds(wid * 16, 16)])


def kernel(vertices, opt_d_rotations, opt_d_translations, weights, nodes_idx,
           influence_nodes_idx, one_ring_neigh):
    f32 = jnp.float32
    i32 = jnp.int32
    vp = jnp.zeros((3, VP), f32).at[:, :NV].set(vertices.T).reshape(-1)
    wp = jnp.zeros((3, VP), f32).at[:, :NV].set(weights.T).reshape(-1)
    ip = jnp.zeros((3, VP), i32).at[:, :NV].set(
        influence_nodes_idx.T.astype(i32)).reshape(-1)
    nidx = jnp.zeros((NNP,), i32).at[:NN].set(nodes_idx.astype(i32))
    rv = jnp.zeros((3, NNP), f32).at[:, :NN].set(opt_d_rotations[0].T)
    tv = jnp.zeros((3, NNP), f32).at[:, :NN].set(opt_d_translations[0].T)
    ring = jnp.zeros((NWORK * AGRP * 16, NB), i32).at[:NN].set(
        one_ring_neigh.astype(i32))
    ring = ring.reshape(NWORK * AGRP, 16, NB).transpose(0, 2, 1).reshape(-1)

    nplanes = _gather_nodes(vp, nidx)
    table = _tc_table(rv.reshape(3, 8, 128), tv.reshape(3, 8, 128),
                      nplanes.reshape(3, 8, 128))
    warp, loss_part = _warp_arap(vp, wp, ip, table.reshape(-1), ring)
    warped = warp.reshape(3, VP)[:, :NV].T[None]
    arap = jnp.sum(loss_part) / f32(NN)
    return warped, arap


# speedup vs baseline: 1.4900x; 1.0012x over previous
import functools

import jax
import jax.numpy as jnp
from jax import lax
from jax.experimental import pallas as pl
from jax.experimental.pallas import tpu as pltpu
from jax.experimental.pallas import tpu_sc as plsc

NV = 6890
NN = 689
K = 3
NB = 18

NNP = 1024
VP = 7168
NWORK = 32
VPT = VP // NWORK
WGRP = VPT // 16
AGRP = NNP // 16 // NWORK

_mesh = plsc.VectorSubcoreMesh(core_axis_name="c", subcore_axis_name="s")
_sc_params = pltpu.CompilerParams(needs_layout_passes=False)


def _wid():
    return lax.axis_index("s") * 2 + lax.axis_index("c")


@functools.partial(
    pl.kernel,
    mesh=_mesh,
    out_type=jax.ShapeDtypeStruct((3 * NNP,), jnp.float32),
    compiler_params=_sc_params,
    scratch_types=[
        pltpu.VMEM((3 * VP,), jnp.float32),
        pltpu.VMEM((32,), jnp.int32),
        pltpu.VMEM((3 * 32,), jnp.float32),
        pltpu.SemaphoreType.DMA,
    ],
)
def _gather_nodes(v_hbm, idx_hbm, out_hbm, v_v, idx_v, out_v, sem):
    wid = _wid()
    cps = [pltpu.async_copy(v_hbm, v_v, sem),
           pltpu.async_copy(idx_hbm.at[pl.ds(wid * 32, 32)], idx_v, sem)]
    for cp in cps:
        cp.wait()
    for gg in range(2):
        j = idx_v[pl.ds(gg * 16, 16)]
        for ci in range(3):
            out_v[pl.ds(ci * 32 + gg * 16, 16)] = plsc.load_gather(
                v_v, [j + ci * VP])
    ocps = [
        pltpu.async_copy(out_v.at[pl.ds(ci * 32, 32)],
                         out_hbm.at[pl.ds(ci * NNP + wid * 32, 32)], sem)
        for ci in range(3)
    ]
    for cp in ocps:
        cp.wait()


def _tc_body(r_ref, t_ref, n_ref, out_ref):
    x = r_ref[0]
    y = r_ref[1]
    z = r_ref[2]
    xa = x + 1e-8
    ya = y + 1e-8
    za = z + 1e-8
    ang = jnp.sqrt(xa * xa + ya * ya + za * za)
    ax = x / ang
    ay = y / ang
    az = z / ang
    c = jnp.cos(ang)
    s = jnp.sin(ang)
    cc = 1.0 - c
    r00 = c + cc * ax * ax
    r01 = cc * ax * ay - s * az
    r02 = cc * ax * az + s * ay
    r10 = cc * ax * ay + s * az
    r11 = c + cc * ay * ay
    r12 = cc * ay * az - s * ax
    r20 = cc * ax * az - s * ay
    r21 = cc * ay * az + s * ax
    r22 = c + cc * az * az
    nx = n_ref[0]
    ny = n_ref[1]
    nz = n_ref[2]
    pmx = nx + t_ref[0]
    pmy = ny + t_ref[1]
    pmz = nz + t_ref[2]
    out_ref[0] = r00
    out_ref[1] = r01
    out_ref[2] = r02
    out_ref[3] = r10
    out_ref[4] = r11
    out_ref[5] = r12
    out_ref[6] = r20
    out_ref[7] = r21
    out_ref[8] = r22
    out_ref[9] = pmx - (r00 * nx + r01 * ny + r02 * nz)
    out_ref[10] = pmy - (r10 * nx + r11 * ny + r12 * nz)
    out_ref[11] = pmz - (r20 * nx + r21 * ny + r22 * nz)
    out_ref[12] = pmx
    out_ref[13] = pmy
    out_ref[14] = pmz
    out_ref[15] = nx
    out_ref[16] = ny
    out_ref[17] = nz


_tc_table = pl.pallas_call(
    _tc_body,
    out_shape=jax.ShapeDtypeStruct((18, 8, 128), jnp.float32),
)


@functools.partial(
    pl.kernel,
    mesh=_mesh,
    out_type=(
        jax.ShapeDtypeStruct((3 * VP,), jnp.float32),
        jax.ShapeDtypeStruct((NWORK * 16,), jnp.float32),
    ),
    compiler_params=_sc_params,
    scratch_types=[
        pltpu.VMEM((18 * NNP,), jnp.float32),
        pltpu.VMEM((3 * VPT,), jnp.float32),
        pltpu.VMEM((3 * VPT,), jnp.float32),
        pltpu.VMEM((3 * VPT,), jnp.int32),
        pltpu.VMEM((AGRP * NB * 16,), jnp.int32),
        pltpu.VMEM((3 * VPT,), jnp.float32),
        pltpu.VMEM((16,), jnp.float32),
        pltpu.SemaphoreType.DMA,
    ],
)
def _warp_arap(v_hbm, w_hbm, ix_hbm, tab_hbm, ring_hbm, warp_hbm, loss_hbm,
               tab_v, v_v, w_v, ix_v, ring_v, wout_v, loss_v, sem):
    wid = _wid()
    base = wid * VPT
    cps = [pltpu.async_copy(tab_hbm, tab_v, sem)]
    for ci in range(3):
        sl_h = pl.ds(ci * VP + base, VPT)
        sl_v = pl.ds(ci * VPT, VPT)
        cps.append(pltpu.async_copy(v_hbm.at[sl_h], v_v.at[sl_v], sem))
        cps.append(pltpu.async_copy(w_hbm.at[sl_h], w_v.at[sl_v], sem))
        cps.append(pltpu.async_copy(ix_hbm.at[sl_h], ix_v.at[sl_v], sem))
    cps.append(pltpu.async_copy(
        ring_hbm.at[pl.ds(wid * AGRP * NB * 16, AGRP * NB * 16)], ring_v,
        sem))
    for cp in cps:
        cp.wait()

    def warp_group(g, carry):
        vx = v_v[pl.ds(g * 16, 16)]
        vy = v_v[pl.ds(VPT + g * 16, 16)]
        vz = v_v[pl.ds(2 * VPT + g * 16, 16)]
        acc = [jnp.zeros((16,), jnp.float32) for _ in range(12)]
        for k in range(K):
            j = ix_v[pl.ds(k * VPT + g * 16, 16)]
            w = w_v[pl.ds(k * VPT + g * 16, 16)]
            for t in range(12):
                acc[t] = acc[t] + w * plsc.load_gather(tab_v, [j + t * NNP])
        wout_v[pl.ds(g * 16, 16)] = (
            acc[0] * vx + acc[1] * vy + acc[2] * vz + acc[9])
        wout_v[pl.ds(VPT + g * 16, 16)] = (
            acc[3] * vx + acc[4] * vy + acc[5] * vz + acc[10])
        wout_v[pl.ds(2 * VPT + g * 16, 16)] = (
            acc[6] * vx + acc[7] * vy + acc[8] * vz + acc[11])
        return carry

    lax.fori_loop(0, WGRP, warp_group, 0)
    ocps = [
        pltpu.async_copy(wout_v.at[pl.ds(ci * VPT, VPT)],
                         warp_hbm.at[pl.ds(ci * VP + base, VPT)], sem)
        for ci in range(3)
    ]

    ids = lax.iota(jnp.int32, 16)
    acc_loss = jnp.zeros((16,), jnp.float32)
    for gg in range(AGRP):
        gbase = (wid * AGRP + gg) * 16
        r = [tab_v[pl.ds(t * NNP + gbase, 16)] for t in range(9)]
        pm = [tab_v[pl.ds((12 + ci) * NNP + gbase, 16)] for ci in range(3)]
        nn = [tab_v[pl.ds((15 + ci) * NNP + gbase, 16)] for ci in range(3)]
        valid = (gbase + ids) < NN
        for h in range(NB):
            m = ring_v[pl.ds((gg * NB + h) * 16, 16)]
            nm = [plsc.load_gather(tab_v, [m + (15 + ci) * NNP])
                  for ci in range(3)]
            pmm = [plsc.load_gather(tab_v, [m + (12 + ci) * NNP])
                   for ci in range(3)]
            dx = nn[0] - nm[0]
            dy = nn[1] - nm[1]
            dz = nn[2] - nm[2]
            ex = pm[0] - pmm[0] - (r[0] * dx + r[1] * dy + r[2] * dz)
            ey = pm[1] - pmm[1] - (r[3] * dx + r[4] * dy + r[5] * dz)
            ez = pm[2] - pmm[2] - (r[6] * dx + r[7] * dy + r[8] * dz)
            e2 = ex * ex + ey * ey + ez * ez
            acc_loss = acc_loss + jnp.where(valid, e2, 0.0)
    loss_v[...] = acc_loss
    ocps.append(pltpu.async_copy(loss_v, loss_hbm.at[pl.ds(wid * 16, 16)],
                                 sem))
    for cp in ocps:
        cp.wait()


def kernel(vertices, opt_d_rotations, opt_d_translations, weights, nodes_idx,
           influence_nodes_idx, one_ring_neigh):
    f32 = jnp.float32
    i32 = jnp.int32
    vp = jnp.zeros((3, VP), f32).at[:, :NV].set(vertices.T).reshape(-1)
    wp = jnp.zeros((3, VP), f32).at[:, :NV].set(weights.T).reshape(-1)
    ip = jnp.zeros((3, VP), i32).at[:, :NV].set(
        influence_nodes_idx.T.astype(i32)).reshape(-1)
    nidx = jnp.zeros((NNP,), i32).at[:NN].set(nodes_idx.astype(i32))
    rv = jnp.zeros((3, NNP), f32).at[:, :NN].set(opt_d_rotations[0].T)
    tv = jnp.zeros((3, NNP), f32).at[:, :NN].set(opt_d_translations[0].T)
    ring = jnp.zeros((NWORK * AGRP * 16, NB), i32).at[:NN].set(
        one_ring_neigh.astype(i32))
    ring = ring.reshape(NWORK * AGRP, 16, NB).transpose(0, 2, 1).reshape(-1)

    nplanes = _gather_nodes(vp, nidx)
    table = _tc_table(rv.reshape(3, 8, 128), tv.reshape(3, 8, 128),
                      nplanes.reshape(3, 8, 128))
    warp, loss_part = _warp_arap(vp, wp, ip, table.reshape(-1), ring)
    warped = warp.reshape(3, VP)[:, :NV].T[None]
    arap = jnp.sum(loss_part) / f32(NN)
    return warped, arap
